# baseline (speedup 1.0000x reference)
import jax
import jax.numpy as jnp
from jax import lax
from jax.experimental import pallas as pl
from jax.experimental.pallas import tpu as pltpu

N = 4096
F = 784
DM = 256
TOPK = 11
RB = 256
NB = N // RB
INV_SQRT_K = 11.0 ** -0.5
BIG = 3.0e38


def _k1_body(x_blk_ref, x_all_ref, theta_ref, sqb_ref, sqt_ref,
             h_ref, cnt_ref, y_ref):
    i = pl.program_id(0)
    x = x_all_ref[...]
    xb = x_blk_ref[...]
    c = lax.dot_general(xb, x, (((1,), (1,)), ((), ())),
                        preferred_element_type=jnp.float32)
    dist = jnp.abs(sqb_ref[...] + sqt_ref[...] - 2.0 * c)
    col = lax.broadcasted_iota(jnp.int32, (RB, N), 1)
    d = dist
    for _ in range(TOPK):
        m = jnp.min(d, axis=1, keepdims=True)
        idx = jnp.min(jnp.where(d == m, col, N), axis=1, keepdims=True)
        d = jnp.where(col == idx, BIG, d)
    onehots = jnp.where(d == BIG, 1.0, 0.0)
    hb = onehots.T
    h_ref[...] = hb
    cnt = jnp.sum(hb, axis=1, keepdims=True)

    @pl.when(i == 0)
    def _():
        cnt_ref[...] = jnp.zeros_like(cnt_ref)

    cnt_ref[...] += cnt
    y_ref[...] = jnp.dot(xb, theta_ref[...],
                         preferred_element_type=jnp.float32)


def _k2_body(h_ref, y_ref, cnt_ref, xo_ref):
    dv = lax.rsqrt(cnt_ref[...])
    dvy = y_ref[...] * dv
    xo = lax.dot_general(h_ref[...], dvy, (((0,), (0,)), ((), ())),
                         preferred_element_type=jnp.float32)
    xo_ref[...] = xo * INV_SQRT_K


def _k3_body(h_ref, xo_ref, cnt_ref, e_ref):
    acc = jnp.dot(h_ref[...], xo_ref[...],
                  preferred_element_type=jnp.float32)
    dv = lax.rsqrt(cnt_ref[...])
    e_ref[...] = acc * dv * INV_SQRT_K


def _make_k1():
    return pl.pallas_call(
        _k1_body,
        grid=(NB,),
        in_specs=[
            pl.BlockSpec((RB, F), lambda i: (i, 0)),
            pl.BlockSpec((N, F), lambda i: (0, 0)),
            pl.BlockSpec((F, DM), lambda i: (0, 0)),
            pl.BlockSpec((RB, 1), lambda i: (i, 0)),
            pl.BlockSpec((1, N), lambda i: (0, 0)),
        ],
        out_specs=[
            pl.BlockSpec((N, RB), lambda i: (0, i)),
            pl.BlockSpec((N, 1), lambda i: (0, 0)),
            pl.BlockSpec((RB, DM), lambda i: (i, 0)),
        ],
        out_shape=[
            jax.ShapeDtypeStruct((N, N), jnp.float32),
            jax.ShapeDtypeStruct((N, 1), jnp.float32),
            jax.ShapeDtypeStruct((N, DM), jnp.float32),
        ],
    )


def _make_k2():
    return pl.pallas_call(
        _k2_body,
        grid=(NB,),
        in_specs=[
            pl.BlockSpec((N, RB), lambda e: (0, e)),
            pl.BlockSpec((N, DM), lambda e: (0, 0)),
            pl.BlockSpec((N, 1), lambda e: (0, 0)),
        ],
        out_specs=pl.BlockSpec((RB, DM), lambda e: (e, 0)),
        out_shape=jax.ShapeDtypeStruct((N, DM), jnp.float32),
    )


def _make_k3():
    return pl.pallas_call(
        _k3_body,
        grid=(NB,),
        in_specs=[
            pl.BlockSpec((RB, N), lambda r: (r, 0)),
            pl.BlockSpec((N, DM), lambda r: (0, 0)),
            pl.BlockSpec((RB, 1), lambda r: (r, 0)),
        ],
        out_specs=pl.BlockSpec((RB, DM), lambda r: (r, 0)),
        out_shape=jax.ShapeDtypeStruct((N, DM), jnp.float32),
    )


def kernel(X, theta):
    X = X.reshape(-1, F)
    sq = jnp.sum(X ** 2, axis=1)
    H, cnt, Y = _make_k1()(X, X, theta, sq.reshape(N, 1),
                           sq.reshape(1, N))
    Xo = _make_k2()(H, Y, cnt)
    E = _make_k3()(H, Xo, cnt)
    return (Xo, E, H)
